# baseline (speedup 1.0000x reference)
import functools

import jax
import jax.numpy as jnp
from jax import lax
from jax.experimental import pallas as pl
from jax.experimental.pallas import tpu as pltpu
from jax.experimental.pallas import tpu_sc as plsc

_DIM = 32
_LAMBDA = 0.01
_CHUNK = 128


def _sc_stage(h, r, t, ent_re, ent_im, rel_re, rel_im, n_rows):
    info = plsc.get_sparse_core_info()
    nc, ns = info.num_cores, info.num_subcores
    nw = nc * ns
    rows_per_w = n_rows // nw
    n_chunks = rows_per_w // _CHUNK
    mesh = plsc.VectorSubcoreMesh(core_axis_name="c", subcore_axis_name="s")

    @functools.partial(
        pl.kernel,
        mesh=mesh,
        compiler_params=pltpu.CompilerParams(use_tc_tiling_on_sc=False),
        out_type=(
            jax.ShapeDtypeStruct((n_rows, 16), jnp.float32),
            jax.ShapeDtypeStruct((nw, 16), jnp.float32),
        ),
        scratch_types=[
            pltpu.VMEM((_CHUNK,), jnp.int32),
            pltpu.VMEM((_CHUNK,), jnp.int32),
            pltpu.VMEM((_CHUNK,), jnp.int32),
            pltpu.VMEM((_CHUNK, _DIM), jnp.float32),
            pltpu.VMEM((_CHUNK, _DIM), jnp.float32),
            pltpu.VMEM((_CHUNK, _DIM), jnp.float32),
            pltpu.VMEM((_CHUNK, _DIM), jnp.float32),
            pltpu.VMEM((_CHUNK, _DIM), jnp.float32),
            pltpu.VMEM((_CHUNK, _DIM), jnp.float32),
            pltpu.VMEM((_CHUNK, 16), jnp.float32),
            pltpu.VMEM((16,), jnp.float32),
            pltpu.SemaphoreType.DMA,
        ],
    )
    def sc_kernel(h_hbm, r_hbm, t_hbm, ere_hbm, eim_hbm, rre_hbm, rim_hbm,
                  p_out, sq_out,
                  hi_v, ri_v, ti_v, beh, bih, bet, bit_, brr, bri,
                  p_v, sq_v, sem):
        wid = lax.axis_index("s") * nc + lax.axis_index("c")
        base_w = wid * rows_per_w

        sq_v[...] = jnp.zeros((16,), jnp.float32)

        for g in range(n_chunks):
            base = base_w + g * _CHUNK
            pltpu.sync_copy(h_hbm.at[pl.ds(base, _CHUNK)], hi_v)
            pltpu.sync_copy(r_hbm.at[pl.ds(base, _CHUNK)], ri_v)
            pltpu.sync_copy(t_hbm.at[pl.ds(base, _CHUNK)], ti_v)
            d0 = pltpu.async_copy(ere_hbm.at[hi_v], beh, sem)
            d1 = pltpu.async_copy(eim_hbm.at[hi_v], bih, sem)
            d2 = pltpu.async_copy(ere_hbm.at[ti_v], bet, sem)
            d3 = pltpu.async_copy(eim_hbm.at[ti_v], bit_, sem)
            d4 = pltpu.async_copy(rre_hbm.at[ri_v], brr, sem)
            d5 = pltpu.async_copy(rim_hbm.at[ri_v], bri, sem)
            d0.wait(); d1.wait(); d2.wait(); d3.wait(); d4.wait(); d5.wait()

            def body(i, acc):
                reh0 = beh[i, pl.ds(0, 16)]
                reh1 = beh[i, pl.ds(16, 16)]
                imh0 = bih[i, pl.ds(0, 16)]
                imh1 = bih[i, pl.ds(16, 16)]
                ret0 = bet[i, pl.ds(0, 16)]
                ret1 = bet[i, pl.ds(16, 16)]
                imt0 = bit_[i, pl.ds(0, 16)]
                imt1 = bit_[i, pl.ds(16, 16)]
                rre0 = brr[i, pl.ds(0, 16)]
                rre1 = brr[i, pl.ds(16, 16)]
                rim0 = bri[i, pl.ds(0, 16)]
                rim1 = bri[i, pl.ds(16, 16)]
                s0 = rre0 * (reh0 * ret0 + imh0 * imt0) + rim0 * (reh0 * imt0 - imh0 * ret0)
                s1 = rre1 * (reh1 * ret1 + imh1 * imt1) + rim1 * (reh1 * imt1 - imh1 * ret1)
                p_v[i, :] = s0 + s1
                q0 = (reh0 * reh0 + imh0 * imh0 + ret0 * ret0
                      + imt0 * imt0 + rre0 * rre0 + rim0 * rim0)
                q1 = (reh1 * reh1 + imh1 * imh1 + ret1 * ret1
                      + imt1 * imt1 + rre1 * rre1 + rim1 * rim1)
                return acc + q0 + q1

            acc = lax.fori_loop(0, _CHUNK, body, sq_v[...])
            sq_v[...] = acc
            pltpu.sync_copy(p_v, p_out.at[pl.ds(base, _CHUNK)])

        pltpu.sync_copy(sq_v, sq_out.at[wid])

    return sc_kernel(h, r, t, ent_re, ent_im, rel_re, rel_im)


def _tc_reduce(p16, sq, n_rows):
    def body(p_ref, sq_ref, o_ref):
        score = jnp.sum(p_ref[...], axis=1, keepdims=True)
        sp = jnp.maximum(score, 0.0) + jnp.log(1.0 + jnp.exp(-jnp.abs(score)))
        loss = jnp.sum(sp) * (1.0 / n_rows)
        regul = jnp.sum(sq_ref[...]) * (1.0 / (n_rows * _DIM))
        o_ref[0, 0] = loss + _LAMBDA * regul

    out = pl.pallas_call(
        body,
        out_shape=jax.ShapeDtypeStruct((1, 1), jnp.float32),
        out_specs=pl.BlockSpec(memory_space=pltpu.SMEM),
    )(p16, sq)
    return out[0, 0]


def kernel(pos_h, pos_r, pos_t, neg_h, neg_r, neg_t, ent_re, ent_im, rel_re, rel_im):
    h = jnp.concatenate([pos_h, neg_h])
    r = jnp.concatenate([pos_r, neg_r])
    t = jnp.concatenate([pos_t, neg_t])
    n_rows = h.shape[0]
    p16, sq = _sc_stage(h, r, t, ent_re, ent_im, rel_re, rel_im, n_rows)
    return _tc_reduce(p16, sq, n_rows)
